# baseline (speedup 1.0000x reference)
import functools

import jax
import jax.numpy as jnp
from jax import lax
from jax.experimental import pallas as pl
from jax.experimental.pallas import tpu as pltpu
from jax.experimental.pallas import tpu_sc as plsc

NC = 2
NS = 16
NW = NC * NS
L = 16
K = 112
NBUF = 3
MRING = 6


@functools.lru_cache(maxsize=None)
def _sc_scatter(N, D, n_chunks):
    rpt = N // NS
    mesh = plsc.VectorSubcoreMesh(
        core_axis_name="c", subcore_axis_name="s", num_cores=NC, num_subcores=NS
    )

    @functools.partial(
        pl.kernel,
        out_type=jax.ShapeDtypeStruct((NC, N, D), jnp.float32),
        mesh=mesh,
        scratch_types=[
            pltpu.VMEM((MRING, 3, K), jnp.int32),
            pltpu.VMEM((NBUF, K, D), jnp.float32),
            pltpu.VMEM_SHARED((N, D), jnp.float32),
            pltpu.SemaphoreType.DMA,
            pltpu.SemaphoreType.DMA,
            pltpu.SemaphoreType.DMA,
            pltpu.SemaphoreType.DMA,
            pltpu.SemaphoreType.DMA,
            pltpu.SemaphoreType.DMA,
            pltpu.SemaphoreType.DMA,
            pltpu.SemaphoreType.DMA,
            pltpu.SemaphoreType.DMA,
            pltpu.SemaphoreType.DMA,
        ],
    )
    def sc_kernel(prev_hbm, msgs_hbm, zeros_hbm, out_hbm,
                  mr, rows_v, acc,
                  sem_g, ss0, ss1, ss2, sm0, sm1, sm2, sm3, sm4, sm5):
        sem_s = (ss0, ss1, ss2)
        sem_m = (sm0, sm1, sm2, sm3, sm4, sm5)
        c = lax.axis_index("c")
        s = lax.axis_index("s")
        wid = s * NC + c
        cbase = wid * n_chunks

        def msg_copy(cm, slot):
            pltpu.async_copy(msgs_hbm.at[cbase + cm], mr.at[slot],
                             sem_m[slot])

        def wait_msgs(slot):
            pltpu.make_async_copy(msgs_hbm.at[0], mr.at[slot],
                                  sem_m[slot]).wait()

        def wait_gather(b):
            pltpu.make_async_copy(prev_hbm.at[mr.at[0, 0]],
                                  rows_v.at[b], sem_g).wait()

        def wait_scatter(b):
            pltpu.make_async_copy(rows_v.at[b], acc.at[mr.at[0, 1]],
                                  sem_s[b]).wait()

        pltpu.sync_copy(zeros_hbm.at[pl.ds(s * rpt, rpt)],
                        acc.at[pl.ds(s * rpt, rpt)])
        for t in range(4):
            msg_copy(t, t)
        plsc.subcore_barrier()

        wait_msgs(0)
        pltpu.sync_copy(zeros_hbm.at[pl.ds(0, K)], rows_v.at[1])
        pltpu.sync_copy(zeros_hbm.at[pl.ds(0, K)], rows_v.at[2])
        pltpu.async_copy(rows_v.at[1], acc.at[mr.at[0, 1]], ss1, add=True)
        pltpu.async_copy(rows_v.at[2], acc.at[mr.at[0, 1]], ss2, add=True)
        pltpu.async_copy(prev_hbm.at[mr.at[0, 0]], rows_v.at[0], sem_g)

        def scale(b, slot):
            def group_body(gi, carry):
                gb = gi * L
                g16 = lax.bitcast_convert_type(
                    mr[slot, 2, pl.ds(gb, L)], jnp.float32)
                for l in range(L):
                    i = gb + l
                    glane = lax.gather(
                        g16, jnp.full((L, 1), l, jnp.int32),
                        lax.GatherDimensionNumbers(
                            offset_dims=(), collapsed_slice_dims=(0,),
                            start_index_map=(0,)),
                        slice_sizes=(1,),
                        mode=lax.GatherScatterMode.PROMISE_IN_BOUNDS)
                    for jb in range(D // L):
                        sl = pl.ds(jb * L, L)
                        rows_v[b, i, sl] = rows_v[b, i, sl] * glane
                return carry
            lax.fori_loop(0, K // L, group_body, 0)

        def ring_body(kk, carry):
            for j in range(MRING):
                ci = kk * MRING + j
                b = j % NBUF
                nb = (j + 1) % NBUF
                nslot = (j + 1) % MRING
                pslot = (j + 4) % MRING
                wait_gather(b)
                wait_scatter(nb)
                cm = ci + 4
                cmw = jnp.where(cm >= n_chunks, cm - n_chunks, cm)
                msg_copy(cmw, pslot)
                wait_msgs(nslot)
                pltpu.async_copy(prev_hbm.at[mr.at[nslot, 0]],
                                 rows_v.at[nb], sem_g)
                scale(b, j)
                pltpu.async_copy(rows_v.at[b], acc.at[mr.at[j, 1]],
                                 sem_s[b], add=True)
            return carry

        lax.fori_loop(0, n_chunks // MRING, ring_body, 0)
        wait_gather(0)
        wait_scatter(1)
        wait_scatter(2)
        wait_msgs(1)
        wait_msgs(2)
        wait_msgs(3)
        plsc.subcore_barrier()
        pltpu.sync_copy(acc.at[pl.ds(s * rpt, rpt)],
                        out_hbm.at[c, pl.ds(s * rpt, rpt)])

    return sc_kernel


@functools.lru_cache(maxsize=None)
def _tc_finish(N, D, bn):
    def body(p_ref, nfe_ref, efe_ref, w_ref, o_ref):
        x = p_ref[0] + p_ref[1]
        x2 = lax.dot_general(x, w_ref[...], (((1,), (1,)), ((), ())),
                             preferred_element_type=jnp.float32)
        y = nfe_ref[0] + x2 + efe_ref[0]
        o_ref[0] = jnp.where(y >= 0, y, 0.01 * y)

    return pl.pallas_call(
        body,
        grid=(N // bn,),
        in_specs=[
            pl.BlockSpec((2, bn, D), lambda i: (0, i, 0)),
            pl.BlockSpec((1, bn, D), lambda i: (0, i, 0)),
            pl.BlockSpec((1, bn, D), lambda i: (0, i, 0)),
            pl.BlockSpec((D, D), lambda i: (0, 0)),
        ],
        out_specs=pl.BlockSpec((1, bn, D), lambda i: (0, i, 0)),
        out_shape=jax.ShapeDtypeStruct((1, N, D), jnp.float32),
    )


def kernel(prev_embeddings, edges_ij, node_features_embeddings,
           edge_features_embeddings, edge_status, W):
    B, N, D = prev_embeddings.shape
    E = edges_ij.shape[0]
    prev2d = prev_embeddings.reshape(N, D)
    Na = -(-N // (NS * 8)) * (NS * 8)

    u = edges_ij[:, 0]
    v = edges_ij[:, 1]
    g = edge_status.astype(jnp.float32)
    M = 2 * E
    n_chunks = -(-M // (NW * K * MRING)) * MRING
    pad = NW * K * n_chunks - M
    pidx = (jnp.arange(pad, dtype=jnp.int32) * 37) % N
    src = jnp.concatenate([u, v, pidx])
    dst = jnp.concatenate([v, u, pidx])
    gate = jnp.concatenate([g, g, jnp.zeros((pad,), jnp.float32)])
    msgs = jnp.stack(
        [src.reshape(-1, K), dst.reshape(-1, K),
         gate.view(jnp.int32).reshape(-1, K)], axis=1)
    zeros = jnp.zeros((Na, D), jnp.float32)

    parts = _sc_scatter(Na, D, n_chunks)(prev2d, msgs, zeros)
    out = _tc_finish(N, D, 2000)(
        parts, node_features_embeddings, edge_features_embeddings, W)
    return out


# speedup vs baseline: 1.0608x; 1.0608x over previous
import functools

import jax
import jax.numpy as jnp
from jax import lax
from jax.experimental import pallas as pl
from jax.experimental.pallas import tpu as pltpu
from jax.experimental.pallas import tpu_sc as plsc

NC = 2
NS = 16
NW = NC * NS
L = 16
K = 112
NBUF = 3
MRING = 6


@functools.lru_cache(maxsize=None)
def _sc_scatter(N, D, n_chunks):
    rpt = N // NS
    mesh = plsc.VectorSubcoreMesh(
        core_axis_name="c", subcore_axis_name="s", num_cores=NC, num_subcores=NS
    )

    @functools.partial(
        pl.kernel,
        out_type=jax.ShapeDtypeStruct((NC, N, D), jnp.float32),
        mesh=mesh,
        scratch_types=[
            pltpu.VMEM((MRING, K), jnp.int32),
            pltpu.VMEM((MRING, K), jnp.int32),
            pltpu.VMEM((MRING, K), jnp.float32),
            pltpu.VMEM((NBUF, K, D), jnp.float32),
            pltpu.VMEM_SHARED((N, D), jnp.float32),
            pltpu.SemaphoreType.DMA,
            pltpu.SemaphoreType.DMA,
            pltpu.SemaphoreType.DMA,
            pltpu.SemaphoreType.DMA,
            pltpu.SemaphoreType.DMA,
            pltpu.SemaphoreType.DMA,
            pltpu.SemaphoreType.DMA,
            pltpu.SemaphoreType.DMA,
            pltpu.SemaphoreType.DMA,
            pltpu.SemaphoreType.DMA,
        ],
    )
    def sc_kernel(prev_hbm, src_hbm, dst_hbm, gate_hbm, zeros_hbm, out_hbm,
                  src_r, dst_r, gate_r, rows_v, acc,
                  sem_g, ss0, ss1, ss2, sm0, sm1, sm2, sm3, sm4, sm5):
        sem_s = (ss0, ss1, ss2)
        sem_m = (sm0, sm1, sm2, sm3, sm4, sm5)
        c = lax.axis_index("c")
        s = lax.axis_index("s")
        wid = s * NC + c
        base = wid * (n_chunks * K)

        def msg_copy(cm, slot):
            off = base + cm * K
            pltpu.async_copy(src_hbm.at[pl.ds(off, K)],
                             src_r.at[slot], sem_m[slot])
            pltpu.async_copy(dst_hbm.at[pl.ds(off, K)],
                             dst_r.at[slot], sem_m[slot])
            pltpu.async_copy(gate_hbm.at[pl.ds(off, K)],
                             gate_r.at[slot], sem_m[slot])

        def wait_msgs(slot):
            pltpu.make_async_copy(src_hbm.at[pl.ds(0, K)],
                                  src_r.at[slot], sem_m[slot]).wait()
            pltpu.make_async_copy(dst_hbm.at[pl.ds(0, K)],
                                  dst_r.at[slot], sem_m[slot]).wait()
            pltpu.make_async_copy(gate_hbm.at[pl.ds(0, K)],
                                  gate_r.at[slot], sem_m[slot]).wait()

        def wait_gather(b):
            pltpu.make_async_copy(prev_hbm.at[src_r.at[0]],
                                  rows_v.at[b], sem_g).wait()

        def wait_scatter(b):
            pltpu.make_async_copy(rows_v.at[b], acc.at[dst_r.at[0]],
                                  sem_s[b]).wait()

        pltpu.sync_copy(zeros_hbm.at[pl.ds(s * rpt, rpt)],
                        acc.at[pl.ds(s * rpt, rpt)])
        for t in range(4):
            msg_copy(t, t)
        plsc.subcore_barrier()

        wait_msgs(0)
        pltpu.sync_copy(zeros_hbm.at[pl.ds(0, K)], rows_v.at[1])
        pltpu.sync_copy(zeros_hbm.at[pl.ds(0, K)], rows_v.at[2])
        pltpu.async_copy(rows_v.at[1], acc.at[dst_r.at[0]], ss1, add=True)
        pltpu.async_copy(rows_v.at[2], acc.at[dst_r.at[0]], ss2, add=True)
        pltpu.async_copy(prev_hbm.at[src_r.at[0]], rows_v.at[0], sem_g)

        def scale(b, slot):
            def group_body(gi, carry):
                gb = gi * L
                g16 = gate_r[slot, pl.ds(gb, L)]
                for l in range(L):
                    i = gb + l
                    glane = lax.gather(
                        g16, jnp.full((L, 1), l, jnp.int32),
                        lax.GatherDimensionNumbers(
                            offset_dims=(), collapsed_slice_dims=(0,),
                            start_index_map=(0,)),
                        slice_sizes=(1,),
                        mode=lax.GatherScatterMode.PROMISE_IN_BOUNDS)
                    for jb in range(D // L):
                        sl = pl.ds(jb * L, L)
                        rows_v[b, i, sl] = rows_v[b, i, sl] * glane
                return carry
            lax.fori_loop(0, K // L, group_body, 0)

        def ring_body(kk, carry):
            for j in range(MRING):
                ci = kk * MRING + j
                b = j % NBUF
                nb = (j + 1) % NBUF
                nslot = (j + 1) % MRING
                pslot = (j + 4) % MRING
                wait_gather(b)
                wait_scatter(nb)
                cm = ci + 4
                cmw = jnp.where(cm >= n_chunks, cm - n_chunks, cm)
                msg_copy(cmw, pslot)
                wait_msgs(nslot)
                pltpu.async_copy(prev_hbm.at[src_r.at[nslot]],
                                 rows_v.at[nb], sem_g)
                scale(b, j)
                pltpu.async_copy(rows_v.at[b], acc.at[dst_r.at[j]],
                                 sem_s[b], add=True)
            return carry

        lax.fori_loop(0, n_chunks // MRING, ring_body, 0)
        wait_gather(0)
        wait_scatter(1)
        wait_scatter(2)
        wait_msgs(1)
        wait_msgs(2)
        wait_msgs(3)
        plsc.subcore_barrier()
        pltpu.sync_copy(acc.at[pl.ds(s * rpt, rpt)],
                        out_hbm.at[c, pl.ds(s * rpt, rpt)])

    return sc_kernel


@functools.lru_cache(maxsize=None)
def _tc_finish(N, D, bn):
    def body(p_ref, nfe_ref, efe_ref, w_ref, o_ref):
        x = p_ref[0] + p_ref[1]
        x2 = lax.dot_general(x, w_ref[...], (((1,), (1,)), ((), ())),
                             preferred_element_type=jnp.float32)
        y = nfe_ref[0] + x2 + efe_ref[0]
        o_ref[0] = jnp.where(y >= 0, y, 0.01 * y)

    return pl.pallas_call(
        body,
        grid=(N // bn,),
        in_specs=[
            pl.BlockSpec((2, bn, D), lambda i: (0, i, 0)),
            pl.BlockSpec((1, bn, D), lambda i: (0, i, 0)),
            pl.BlockSpec((1, bn, D), lambda i: (0, i, 0)),
            pl.BlockSpec((D, D), lambda i: (0, 0)),
        ],
        out_specs=pl.BlockSpec((1, bn, D), lambda i: (0, i, 0)),
        out_shape=jax.ShapeDtypeStruct((1, N, D), jnp.float32),
    )


def kernel(prev_embeddings, edges_ij, node_features_embeddings,
           edge_features_embeddings, edge_status, W):
    B, N, D = prev_embeddings.shape
    E = edges_ij.shape[0]
    prev2d = prev_embeddings.reshape(N, D)
    Na = -(-N // (NS * 8)) * (NS * 8)

    u = edges_ij[:, 0]
    v = edges_ij[:, 1]
    g = edge_status.astype(jnp.float32)
    M = 2 * E
    n_chunks = -(-M // (NW * K * MRING)) * MRING
    pad = NW * K * n_chunks - M
    pidx = (jnp.arange(pad, dtype=jnp.int32) * 37) % N
    src = jnp.concatenate([u, v, pidx])
    dst = jnp.concatenate([v, u, pidx])
    gate = jnp.concatenate([g, g, jnp.zeros((pad,), jnp.float32)])
    zeros = jnp.zeros((Na, D), jnp.float32)

    parts = _sc_scatter(Na, D, n_chunks)(prev2d, src, dst, gate, zeros)
    out = _tc_finish(N, D, 2000)(
        parts, node_features_embeddings, edge_features_embeddings, W)
    return out
